# baseline (speedup 1.0000x reference)
import jax
import jax.numpy as jnp
import numpy as np
from jax.experimental import pallas as pl
from jax.experimental.pallas import tpu as pltpu
from jax.sharding import Mesh, PartitionSpec as P

_P = jax.lax.Precision.DEFAULT


def _layer_kernel(a_ref, x_ref, w_ref, b_ref, o_ref, s_ref):
    @pl.when(pl.program_id(0) == 0)
    def _():
        s_ref[...] = jnp.dot(
            x_ref[...], w_ref[...],
            preferred_element_type=jnp.float32, precision=_P)

    acc = jnp.dot(
        a_ref[...], s_ref[...],
        preferred_element_type=jnp.float32, precision=_P)
    o_ref[...] = jnp.maximum(acc + b_ref[...], 0.0)


def _gc_layer(a, x, w, b, block_m):
    m, n = a.shape
    k = x.shape[1]
    f = w.shape[1]
    return pl.pallas_call(
        _layer_kernel,
        grid=(m // block_m,),
        in_specs=[
            pl.BlockSpec((block_m, n), lambda i: (i, 0)),
            pl.BlockSpec((n, k), lambda i: (0, 0)),
            pl.BlockSpec((k, f), lambda i: (0, 0)),
            pl.BlockSpec((1, f), lambda i: (0, 0)),
        ],
        out_specs=pl.BlockSpec((block_m, f), lambda i: (i, 0)),
        out_shape=jax.ShapeDtypeStruct((m, f), jnp.float32),
        scratch_shapes=[pltpu.VMEM((n, f), jnp.float32)],
    )(a, x, w, b)


def _gcn3(x, adj_s, a2_s, W3, b3, W1, b1, W2, b2):
    out1 = _gc_layer(adj_s, x, W3, b3, 256)
    out1 = jax.lax.all_gather(out1, "d", axis=0, tiled=True)
    out2 = _gc_layer(a2_s, out1, W1, b1, 256)
    out2 = jax.lax.all_gather(out2, "d", axis=0, tiled=True)
    return _gc_layer(a2_s, out2, W2, b2, 256)


@jax.jit
def kernel(x, adj, A2, W3, b3, W1, b1, W2, b2):
    n_dev = 2 if len(jax.devices()) >= 2 else 1
    mesh = Mesh(np.array(jax.devices()[:n_dev]), ("d",))
    rep = P(None, None)
    fn = jax.shard_map(
        _gcn3,
        mesh=mesh,
        in_specs=(rep, P("d", None), P("d", None),
                  rep, rep, rep, rep, rep, rep),
        out_specs=P("d", None),
        check_vma=False,
    )
    return fn(x, adj, A2,
              W3, b3.reshape(1, -1), W1, b1.reshape(1, -1),
              W2, b2.reshape(1, -1))


# speedup vs baseline: 7.3651x; 7.3651x over previous
import jax
import jax.numpy as jnp
from jax.experimental import pallas as pl
from jax.experimental.pallas import tpu as pltpu

_BM = 256
_NBLK = 4096 // _BM
_BF = jnp.bfloat16


def _layer1_kernel(a_ref, x_ref, w_ref, b_ref, o_ref, s_ref):
    @pl.when(pl.program_id(0) == 0)
    def _():
        sup = jnp.dot(x_ref[...].astype(_BF), w_ref[...].astype(_BF),
                      preferred_element_type=jnp.float32)
        s_ref[...] = sup.astype(_BF)

    acc = jnp.dot(a_ref[...].astype(_BF), s_ref[...],
                  preferred_element_type=jnp.float32)
    o_ref[...] = jnp.maximum(acc + b_ref[...], 0.0).astype(o_ref.dtype)


def _gc_layer1(a, x, w, b):
    n = a.shape[0]
    k = x.shape[1]
    f = w.shape[1]
    return pl.pallas_call(
        _layer1_kernel,
        grid=(n // _BM,),
        in_specs=[
            pl.BlockSpec((_BM, n), lambda i: (i, 0)),
            pl.BlockSpec((n, k), lambda i: (0, 0)),
            pl.BlockSpec((k, f), lambda i: (0, 0)),
            pl.BlockSpec((1, f), lambda i: (0, 0)),
        ],
        out_specs=pl.BlockSpec((_BM, f), lambda i: (i, 0)),
        out_shape=jax.ShapeDtypeStruct((n, f), _BF),
        scratch_shapes=[pltpu.VMEM((n, f), _BF)],
    )(a, x, w, b)


def _layer23_kernel(a_ref, h_ref, w1_ref, b1_ref, w2_ref, b2_ref, o_ref,
                    a2c_ref, s2_ref, s3_ref, h2_ref):
    i = pl.program_id(0)

    @pl.when(i == 0)
    def _():
        sup = jnp.dot(h_ref[...], w1_ref[...].astype(_BF),
                      preferred_element_type=jnp.float32)
        s2_ref[...] = sup.astype(_BF)

    @pl.when(i < _NBLK)
    def _():
        abf = a_ref[...].astype(_BF)
        a2c_ref[pl.ds(i * _BM, _BM), :] = abf
        acc = jnp.dot(abf, s2_ref[...], preferred_element_type=jnp.float32)
        h2 = jnp.maximum(acc + b1_ref[...], 0.0)
        h2_ref[pl.ds(i * _BM, _BM), :] = h2.astype(_BF)

    @pl.when(i == _NBLK)
    def _():
        sup = jnp.dot(h2_ref[...], w2_ref[...].astype(_BF),
                      preferred_element_type=jnp.float32)
        s3_ref[...] = sup.astype(_BF)

    @pl.when(i >= _NBLK)
    def _():
        blk = i - _NBLK
        acc = jnp.dot(a2c_ref[pl.ds(blk * _BM, _BM), :], s3_ref[...],
                      preferred_element_type=jnp.float32)
        o_ref[...] = jnp.maximum(acc + b2_ref[...], 0.0)


def _gc_layers23(a, h, w1, b1, w2, b2):
    n = a.shape[0]
    k = h.shape[1]
    f1 = w1.shape[1]
    f2 = w2.shape[1]
    return pl.pallas_call(
        _layer23_kernel,
        grid=(2 * _NBLK,),
        in_specs=[
            pl.BlockSpec((_BM, n), lambda i: (jnp.minimum(i, _NBLK - 1), 0)),
            pl.BlockSpec((n, k), lambda i: (0, 0)),
            pl.BlockSpec((k, f1), lambda i: (0, 0)),
            pl.BlockSpec((1, f1), lambda i: (0, 0)),
            pl.BlockSpec((f1, f2), lambda i: (0, 0)),
            pl.BlockSpec((1, f2), lambda i: (0, 0)),
        ],
        out_specs=pl.BlockSpec(
            (_BM, f2), lambda i: (jnp.maximum(i - _NBLK, 0), 0)),
        out_shape=jax.ShapeDtypeStruct((n, f2), jnp.float32),
        scratch_shapes=[
            pltpu.VMEM((n, n), _BF),
            pltpu.VMEM((n, f1), _BF),
            pltpu.VMEM((n, f2), _BF),
            pltpu.VMEM((n, f1), _BF),
        ],
    )(a, h, w1, b1, w2, b2)


@jax.jit
def kernel(x, adj, A2, W3, b3, W1, b1, W2, b2):
    out1 = _gc_layer1(adj, x, W3, b3.reshape(1, -1))
    return _gc_layers23(A2, out1, W1, b1.reshape(1, -1),
                        W2, b2.reshape(1, -1))


# speedup vs baseline: 7.7366x; 1.0504x over previous
import jax
import jax.numpy as jnp
from jax.experimental import pallas as pl
from jax.experimental.pallas import tpu as pltpu

_BM = 256
_NBLK = 4096 // _BM
_BF = jnp.bfloat16


def _layer1_kernel(a_ref, x_ref, w_ref, b_ref, o_ref, xc_ref, wc_ref):
    @pl.when(pl.program_id(0) == 0)
    def _():
        xc_ref[...] = x_ref[...].astype(_BF)
        wc_ref[...] = w_ref[...].astype(_BF)

    t = jnp.dot(a_ref[...].astype(_BF), xc_ref[...],
                preferred_element_type=jnp.float32)
    u = jnp.dot(t.astype(_BF), wc_ref[...],
                preferred_element_type=jnp.float32)
    o_ref[...] = jnp.maximum(u + b_ref[...], 0.0).astype(o_ref.dtype)


def _gc_layer1(a, x, w, b):
    n = a.shape[0]
    k = x.shape[1]
    f = w.shape[1]
    return pl.pallas_call(
        _layer1_kernel,
        grid=(n // _BM,),
        in_specs=[
            pl.BlockSpec((_BM, n), lambda i: (i, 0)),
            pl.BlockSpec((n, k), lambda i: (0, 0)),
            pl.BlockSpec((k, f), lambda i: (0, 0)),
            pl.BlockSpec((1, f), lambda i: (0, 0)),
        ],
        out_specs=pl.BlockSpec((_BM, f), lambda i: (i, 0)),
        out_shape=jax.ShapeDtypeStruct((n, f), _BF),
        scratch_shapes=[pltpu.VMEM((n, k), _BF), pltpu.VMEM((k, f), _BF)],
    )(a, x, w, b)


def _layer23_kernel(a_ref, h_ref, w1_ref, b1_ref, w2_ref, b2_ref, o_ref,
                    a2c_ref, s2_ref, s3_ref, h2_ref):
    i = pl.program_id(0)

    @pl.when(i == 0)
    def _():
        sup = jnp.dot(h_ref[...], w1_ref[...].astype(_BF),
                      preferred_element_type=jnp.float32)
        s2_ref[...] = sup.astype(_BF)

    @pl.when(i < _NBLK)
    def _():
        abf = a_ref[...].astype(_BF)
        a2c_ref[pl.ds(i * _BM, _BM), :] = abf
        acc = jnp.dot(abf, s2_ref[...], preferred_element_type=jnp.float32)
        h2 = jnp.maximum(acc + b1_ref[...], 0.0)
        h2_ref[pl.ds(i * _BM, _BM), :] = h2.astype(_BF)

    @pl.when(i == _NBLK)
    def _():
        sup = jnp.dot(h2_ref[...], w2_ref[...].astype(_BF),
                      preferred_element_type=jnp.float32)
        s3_ref[...] = sup.astype(_BF)

    @pl.when(i >= _NBLK)
    def _():
        blk = i - _NBLK
        acc = jnp.dot(a2c_ref[pl.ds(blk * _BM, _BM), :], s3_ref[...],
                      preferred_element_type=jnp.float32)
        o_ref[...] = jnp.maximum(acc + b2_ref[...], 0.0)


def _gc_layers23(a, h, w1, b1, w2, b2):
    n = a.shape[0]
    k = h.shape[1]
    f1 = w1.shape[1]
    f2 = w2.shape[1]
    return pl.pallas_call(
        _layer23_kernel,
        grid=(2 * _NBLK,),
        in_specs=[
            pl.BlockSpec((_BM, n), lambda i: (jnp.minimum(i, _NBLK - 1), 0)),
            pl.BlockSpec((n, k), lambda i: (0, 0)),
            pl.BlockSpec((k, f1), lambda i: (0, 0)),
            pl.BlockSpec((1, f1), lambda i: (0, 0)),
            pl.BlockSpec((f1, f2), lambda i: (0, 0)),
            pl.BlockSpec((1, f2), lambda i: (0, 0)),
        ],
        out_specs=pl.BlockSpec(
            (_BM, f2), lambda i: (jnp.maximum(i - _NBLK, 0), 0)),
        out_shape=jax.ShapeDtypeStruct((n, f2), jnp.float32),
        scratch_shapes=[
            pltpu.VMEM((n, n), _BF),
            pltpu.VMEM((n, f1), _BF),
            pltpu.VMEM((n, f2), _BF),
            pltpu.VMEM((n, f1), _BF),
        ],
    )(a, h, w1, b1, w2, b2)


@jax.jit
def kernel(x, adj, A2, W3, b3, W1, b1, W2, b2):
    out1 = _gc_layer1(adj, x, W3, b3.reshape(1, -1))
    return _gc_layers23(A2, out1, W1, b1.reshape(1, -1),
                        W2, b2.reshape(1, -1))


# speedup vs baseline: 7.8803x; 1.0186x over previous
import jax
import jax.numpy as jnp
from jax.experimental import pallas as pl
from jax.experimental.pallas import tpu as pltpu

_N = 4096
_BM = 256
_NBLK = _N // _BM
_BF = jnp.bfloat16


def _gcn_kernel(adj_ref, a2_ref, x_ref, w3_ref, b3_ref, w1_ref, b1_ref,
                w2_ref, b2_ref, o_ref,
                a2c_ref, xc_ref, w3c_ref, w1c_ref, w2c_ref, s2_ref, s3_ref):
    i = pl.program_id(0)

    @pl.when(i == 0)
    def _():
        xc_ref[...] = x_ref[...].astype(_BF)
        w3c_ref[...] = w3_ref[...].astype(_BF)
        w1c_ref[...] = w1_ref[...].astype(_BF)
        w2c_ref[...] = w2_ref[...].astype(_BF)

    @pl.when(i < _NBLK)
    def _():
        t = jnp.dot(adj_ref[...].astype(_BF), xc_ref[...],
                    preferred_element_type=jnp.float32)
        u = jnp.dot(t.astype(_BF), w3c_ref[...],
                    preferred_element_type=jnp.float32)
        h1 = jnp.maximum(u + b3_ref[...], 0.0).astype(_BF)
        s2 = jnp.dot(h1, w1c_ref[...], preferred_element_type=jnp.float32)
        s2_ref[pl.ds(i * _BM, _BM), :] = s2.astype(_BF)

    @pl.when((i >= _NBLK) & (i < 2 * _NBLK))
    def _():
        blk = i - _NBLK
        abf = a2_ref[...].astype(_BF)
        a2c_ref[pl.ds(blk * _BM, _BM), :] = abf
        acc = jnp.dot(abf, s2_ref[...], preferred_element_type=jnp.float32)
        h2 = jnp.maximum(acc + b1_ref[...], 0.0).astype(_BF)
        s3 = jnp.dot(h2, w2c_ref[...], preferred_element_type=jnp.float32)
        s3_ref[pl.ds(blk * _BM, _BM), :] = s3.astype(_BF)

    @pl.when(i >= 2 * _NBLK)
    def _():
        blk = i - 2 * _NBLK
        acc = jnp.dot(a2c_ref[pl.ds(blk * _BM, _BM), :], s3_ref[...],
                      preferred_element_type=jnp.float32)
        o_ref[...] = jnp.maximum(acc + b2_ref[...], 0.0)


def _adj_map(i):
    return (jnp.minimum(i, _NBLK - 1), 0)


def _a2_map(i):
    return (jnp.clip(i - _NBLK, 0, _NBLK - 1), 0)


def _out_map(i):
    return (jnp.maximum(i - 2 * _NBLK, 0), 0)


_zero_map = lambda i: (0, 0)


@jax.jit
def kernel(x, adj, A2, W3, b3, W1, b1, W2, b2):
    nfeat = W3.shape[1]
    nhid = W1.shape[1]
    ncls = W2.shape[1]
    return pl.pallas_call(
        _gcn_kernel,
        grid=(3 * _NBLK,),
        in_specs=[
            pl.BlockSpec((_BM, _N), _adj_map),
            pl.BlockSpec((_BM, _N), _a2_map),
            pl.BlockSpec((_N, ncls), _zero_map),
            pl.BlockSpec((ncls, nfeat), _zero_map),
            pl.BlockSpec((1, nfeat), _zero_map),
            pl.BlockSpec((nfeat, nhid), _zero_map),
            pl.BlockSpec((1, nhid), _zero_map),
            pl.BlockSpec((nhid, ncls), _zero_map),
            pl.BlockSpec((1, ncls), _zero_map),
        ],
        out_specs=pl.BlockSpec((_BM, ncls), _out_map),
        out_shape=jax.ShapeDtypeStruct((_N, ncls), jnp.float32),
        scratch_shapes=[
            pltpu.VMEM((_N, _N), _BF),
            pltpu.VMEM((_N, ncls), _BF),
            pltpu.VMEM((ncls, nfeat), _BF),
            pltpu.VMEM((nfeat, nhid), _BF),
            pltpu.VMEM((nhid, ncls), _BF),
            pltpu.VMEM((_N, nhid), _BF),
            pltpu.VMEM((_N, ncls), _BF),
        ],
    )(adj, A2, x, W3, b3.reshape(1, -1), W1, b1.reshape(1, -1),
      W2, b2.reshape(1, -1))
